# Initial kernel scaffold; baseline (speedup 1.0000x reference)
import jax
import jax.numpy as jnp
from jax.experimental import pallas as pl


def kernel(z, embeddings):
    raise NotImplementedError("write your pallas kernel here")


# speedup vs baseline: 1.1762x; 1.1762x over previous
import functools

import jax
import jax.numpy as jnp
from jax import lax
from jax.experimental import pallas as pl
from jax.experimental.pallas import tpu as pltpu
from jax.experimental.pallas import tpu_sc as plsc

_B = 100000
_D = 128
_L = 16
_NW = 32
_CHUNK = 448
_NCHUNK = 7
_BPW = _CHUNK * _NCHUNK
_BP = _NW * _BPW

_mesh = plsc.VectorSubcoreMesh(core_axis_name="c", subcore_axis_name="s")


@functools.partial(
    pl.kernel,
    out_type=jax.ShapeDtypeStruct((_BP, _D), jnp.float32),
    mesh=_mesh,
    scratch_types=[
        pltpu.VMEM((_CHUNK,), jnp.int32),
        pltpu.VMEM((_CHUNK, _D), jnp.float32),
        pltpu.SemaphoreType.DMA,
    ],
)
def _sc_gather(z_hbm, table_hbm, out_hbm, idx_v, rows_v, sem):
    wid = lax.axis_index("s") * 2 + lax.axis_index("c")
    base = wid * _BPW

    def chunk_body(ci, carry):
        off = base + ci * _CHUNK
        pltpu.sync_copy(z_hbm.at[pl.ds(off, _CHUNK)], idx_v)

        def sub1(j, c):
            idx_v[pl.ds(j * _L, _L)] = idx_v[pl.ds(j * _L, _L)] - 1
            return c

        lax.fori_loop(0, _CHUNK // _L, sub1, 0)
        pltpu.async_copy(table_hbm.at[idx_v], rows_v, sem).wait()
        pltpu.sync_copy(rows_v, out_hbm.at[pl.ds(off, _CHUNK)])
        return carry

    lax.fori_loop(0, _NCHUNK, chunk_body, 0)


def kernel(z, embeddings):
    zp = jnp.pad(z, (0, _BP - _B), constant_values=1)
    out = _sc_gather(zp, embeddings)
    return out[:_B]


# speedup vs baseline: 1.4760x; 1.2550x over previous
import functools

import jax
import jax.numpy as jnp
from jax import lax
from jax.experimental import pallas as pl
from jax.experimental.pallas import tpu as pltpu
from jax.experimental.pallas import tpu_sc as plsc

_B = 100000
_D = 128
_L = 16
_NW = 32
_CHUNK = 448
_NCHUNK = 7
_BPW = _CHUNK * _NCHUNK

_mesh = plsc.VectorSubcoreMesh(core_axis_name="c", subcore_axis_name="s")


@functools.partial(
    pl.kernel,
    out_type=jax.ShapeDtypeStruct((_B, _D), jnp.float32),
    mesh=_mesh,
    scratch_types=[
        pltpu.VMEM((_BPW,), jnp.int32),
        pltpu.VMEM((2, _CHUNK, _D), jnp.float32),
        pltpu.SemaphoreType.DMA,
        pltpu.SemaphoreType.DMA,
        pltpu.SemaphoreType.DMA,
    ],
)
def _sc_gather(z_hbm, table_hbm, out_hbm, idx_v, rows_v, gsem, ssem0, ssem1):
    wid = lax.axis_index("s") * 2 + lax.axis_index("c")
    base = jnp.minimum(wid * _BPW, _B - _BPW)
    pltpu.sync_copy(z_hbm.at[pl.ds(base, _BPW)], idx_v)

    def sub_chunk(ci):
        def sub1(j, c):
            s = ci * _CHUNK + j * _L
            idx_v[pl.ds(s, _L)] = idx_v[pl.ds(s, _L)] - 1
            return c

        lax.fori_loop(0, _CHUNK // _L, sub1, 0)

    ssems = (ssem0, ssem1)
    sub_chunk(0)
    gather = pltpu.async_copy(
        table_hbm.at[idx_v.at[pl.ds(0, _CHUNK)]], rows_v.at[0], gsem
    )
    stores = [None, None]
    for ci in range(_NCHUNK):
        b = ci % 2
        if ci + 1 < _NCHUNK:
            sub_chunk(ci + 1)
        gather.wait()
        stores[b] = pltpu.async_copy(
            rows_v.at[b], out_hbm.at[pl.ds(base + ci * _CHUNK, _CHUNK)], ssems[b]
        )
        if ci + 1 < _NCHUNK:
            nb = (ci + 1) % 2
            if stores[nb] is not None:
                stores[nb].wait()
            gather = pltpu.async_copy(
                table_hbm.at[idx_v.at[pl.ds((ci + 1) * _CHUNK, _CHUNK)]],
                rows_v.at[nb],
                gsem,
            )
    stores[0].wait()
    stores[1].wait()


def kernel(z, embeddings):
    return _sc_gather(z, embeddings)


# speedup vs baseline: 5.3766x; 3.6425x over previous
import functools

import jax
import jax.numpy as jnp
from jax import lax
from jax.experimental import pallas as pl
from jax.experimental.pallas import tpu as pltpu
from jax.experimental.pallas import tpu_sc as plsc

_B = 100000
_D = 128
_E = 94
_L = 16
_NW = 32
_CHUNK = 112
_NCHUNK = 28
_BPW = _CHUNK * _NCHUNK

_mesh = plsc.VectorSubcoreMesh(core_axis_name="c", subcore_axis_name="s")


@functools.partial(
    pl.kernel,
    out_type=jax.ShapeDtypeStruct((_B, _D), jnp.float32),
    mesh=_mesh,
    scratch_types=[
        pltpu.VMEM((_BPW,), jnp.int32),
        pltpu.VMEM_SHARED((_E, _D), jnp.float32),
        pltpu.VMEM((2, _CHUNK, _D), jnp.float32),
        pltpu.SemaphoreType.DMA,
        pltpu.SemaphoreType.DMA,
        pltpu.SemaphoreType.DMA,
    ],
)
def _sc_gather(z_hbm, table_hbm, out_hbm, idx_v, table_v, rows_v,
               gsem, ssem0, ssem1):
    wid = lax.axis_index("s") * 2 + lax.axis_index("c")
    base = jnp.minimum(wid * _BPW, _B - _BPW)

    @pl.when(lax.axis_index("s") == 0)
    def _stage_table():
        pltpu.sync_copy(table_hbm, table_v)

    pltpu.sync_copy(z_hbm.at[pl.ds(base, _BPW)], idx_v)
    plsc.subcore_barrier()

    def sub_chunk(ci):
        def sub1(j, c):
            s = ci * _CHUNK + j * _L
            idx_v[pl.ds(s, _L)] = idx_v[pl.ds(s, _L)] - 1
            return c

        lax.fori_loop(0, _CHUNK // _L, sub1, 0)

    ssems = (ssem0, ssem1)
    sub_chunk(0)
    gather = pltpu.async_copy(
        table_v.at[idx_v.at[pl.ds(0, _CHUNK)]], rows_v.at[0], gsem
    )
    stores = [None, None]
    for ci in range(_NCHUNK):
        b = ci % 2
        if ci + 1 < _NCHUNK:
            sub_chunk(ci + 1)
        gather.wait()
        stores[b] = pltpu.async_copy(
            rows_v.at[b], out_hbm.at[pl.ds(base + ci * _CHUNK, _CHUNK)], ssems[b]
        )
        if ci + 1 < _NCHUNK:
            nb = (ci + 1) % 2
            if stores[nb] is not None:
                stores[nb].wait()
            gather = pltpu.async_copy(
                table_v.at[idx_v.at[pl.ds((ci + 1) * _CHUNK, _CHUNK)]],
                rows_v.at[nb],
                gsem,
            )
    stores[0].wait()
    stores[1].wait()


def kernel(z, embeddings):
    return _sc_gather(z, embeddings)


# speedup vs baseline: 5.6617x; 1.0530x over previous
import functools

import jax
import jax.numpy as jnp
from jax import lax
from jax.experimental import pallas as pl
from jax.experimental.pallas import tpu as pltpu
from jax.experimental.pallas import tpu_sc as plsc

_B = 100000
_D = 128
_E = 94
_L = 16
_NW = 32
_CHUNK = 112
_NCHUNK = 28
_BPW = _CHUNK * _NCHUNK
_NB = 3

_mesh = plsc.VectorSubcoreMesh(core_axis_name="c", subcore_axis_name="s")


@functools.partial(
    pl.kernel,
    out_type=jax.ShapeDtypeStruct((_B, _D), jnp.float32),
    mesh=_mesh,
    scratch_types=[
        pltpu.VMEM((_BPW,), jnp.int32),
        pltpu.VMEM_SHARED((_E, _D), jnp.float32),
        pltpu.VMEM((_NB, _CHUNK, _D), jnp.float32),
        pltpu.SemaphoreType.DMA,
        pltpu.SemaphoreType.DMA,
        pltpu.SemaphoreType.DMA,
        pltpu.SemaphoreType.DMA,
        pltpu.SemaphoreType.DMA,
        pltpu.SemaphoreType.DMA,
    ],
)
def _sc_gather(z_hbm, table_hbm, out_hbm, idx_v, table_v, rows_v,
               gsem0, gsem1, gsem2, ssem0, ssem1, ssem2):
    wid = lax.axis_index("s") * 2 + lax.axis_index("c")
    base = jnp.minimum(wid * _BPW, _B - _BPW)

    @pl.when(lax.axis_index("s") == 0)
    def _stage_table():
        pltpu.sync_copy(table_hbm, table_v)

    pltpu.sync_copy(z_hbm.at[pl.ds(base, _BPW)], idx_v)
    plsc.subcore_barrier()

    def sub_chunk(ci):
        def sub1(j, c):
            s = ci * _CHUNK + j * _L
            idx_v[pl.ds(s, _L)] = idx_v[pl.ds(s, _L)] - 1
            return c

        lax.fori_loop(0, _CHUNK // _L, sub1, 0)

    gsems = (gsem0, gsem1, gsem2)
    ssems = (ssem0, ssem1, ssem2)

    def issue_gather(ci):
        return pltpu.async_copy(
            table_v.at[idx_v.at[pl.ds(ci * _CHUNK, _CHUNK)]],
            rows_v.at[ci % _NB],
            gsems[ci % _NB],
        )

    def issue_store(ci):
        return pltpu.async_copy(
            rows_v.at[ci % _NB],
            out_hbm.at[pl.ds(base + ci * _CHUNK, _CHUNK)],
            ssems[ci % _NB],
        )

    gathers = {}
    stores = [None] * _NB
    for ci in range(min(_NB - 1, _NCHUNK)):
        sub_chunk(ci)
        gathers[ci] = issue_gather(ci)
    for ci in range(_NCHUNK):
        nxt = ci + _NB - 1
        if nxt < _NCHUNK:
            if stores[nxt % _NB] is not None:
                stores[nxt % _NB].wait()
            sub_chunk(nxt)
            gathers[nxt] = issue_gather(nxt)
        gathers[ci].wait()
        stores[ci % _NB] = issue_store(ci)
    for b in range(_NB):
        if stores[b] is not None:
            stores[b].wait()


def kernel(z, embeddings):
    return _sc_gather(z, embeddings)
